# baseline (speedup 1.0000x reference)
import functools

import jax
import jax.numpy as jnp
from jax import lax
from jax.experimental import pallas as pl
from jax.experimental.pallas import tpu as pltpu
from jax.experimental.pallas import tpu_sc as plsc

N = 10000
E = 320000
D = 128
H = 16
C = 7

NP = 10240
GROUP = 128
NSC = 2
NTILE = 16
NGROUPS = 2560
EP = NGROUPS * GROUP
GP_TILE = NGROUPS // (NSC * NTILE)
ROWS_TILE = NP // NTILE
PAD_DST = NP - 8
ROWS_BLK = 1024
EPS = 1e-16
HW = 32
GB = 16
NBUF = 4
NCHUNK = 4
NPART = NCHUNK * NSC
EB = GB * GROUP // 4


def _pre_body(x_ref, w_ref, b_ref, t_ref):
    h = jnp.dot(x_ref[...], w_ref[...], preferred_element_type=jnp.float32)
    h = jnp.maximum(h + b_ref[...], 0.0)
    nrm = jnp.maximum(jnp.sqrt(jnp.sum(h * h, axis=1)), 1e-12)
    t_ref[0] = h
    t_ref[1] = h / nrm[:, None]


_pre = pl.pallas_call(
    _pre_body,
    grid=(NP // ROWS_BLK,),
    in_specs=[
        pl.BlockSpec((ROWS_BLK, D), lambda i: (i, 0)),
        pl.BlockSpec((D, H), lambda i: (0, 0)),
        pl.BlockSpec((1, H), lambda i: (0, 0)),
    ],
    out_specs=pl.BlockSpec((2, ROWS_BLK, H), lambda i: (0, i, 0)),
    out_shape=jax.ShapeDtypeStruct((2, NP, H), jnp.float32),
)


def _edge_body(hs_ref, hd_ref, beta_ref, out_ref):
    hs = hs_ref[...]
    hd = hd_ref[...]
    bi = lax.broadcasted_iota(jnp.int32, (4 * H, 8 * H), 0)
    bl = lax.broadcasted_iota(jnp.int32, (4 * H, 8 * H), 1)
    seg = (bi // H) == (bl // HW)
    bd = jnp.where(seg, 1.0, 0.0)
    mm = jnp.where(seg & ((bi % H) == (bl % HW)), 1.0, 0.0)
    dot = jnp.dot(hs * hd, bd, preferred_element_type=jnp.float32)
    nsq = jnp.dot(hs * hs, bd, preferred_element_type=jnp.float32)
    nsrc = jnp.maximum(jnp.sqrt(nsq), 1e-12)
    ex = jnp.exp(beta_ref[0, 0] * dot / nsrc)
    hs128 = jnp.dot(hs, mm, preferred_element_type=jnp.float32)
    lane = lax.broadcasted_iota(jnp.int32, ex.shape, 1) % HW
    out_ref[...] = jnp.where(lane < H, hs128 * ex,
                             jnp.where(lane == H, ex, 0.0))


@functools.cache
def _make_edge(ngroups):
    ne = ngroups * GROUP
    return pl.pallas_call(
        _edge_body,
        grid=(ngroups // GB,),
        in_specs=[
            pl.BlockSpec((EB, 4 * H), lambda i: (i, 0)),
            pl.BlockSpec((EB, 4 * H), lambda i: (i, 0)),
            pl.BlockSpec((1, 128), lambda i: (0, 0)),
        ],
        out_specs=pl.BlockSpec((EB, 4 * HW), lambda i: (i, 0)),
        out_shape=jax.ShapeDtypeStruct((ne // 4, 4 * HW), jnp.float32),
    )


def _mid_body(num_ref, t_ref):
    s = jnp.sum(num_ref[...], axis=0)
    h = s[:, :H] / (s[:, H:H + 1] + EPS)
    nrm = jnp.maximum(jnp.sqrt(jnp.sum(h * h, axis=1)), 1e-12)
    t_ref[0] = h
    t_ref[1] = h / nrm[:, None]


_mid = pl.pallas_call(
    _mid_body,
    grid=(NP // ROWS_BLK,),
    in_specs=[
        pl.BlockSpec((NPART, ROWS_BLK, HW), lambda i: (0, i, 0)),
    ],
    out_specs=pl.BlockSpec((2, ROWS_BLK, H), lambda i: (0, i, 0)),
    out_shape=jax.ShapeDtypeStruct((2, NP, H), jnp.float32),
)


def _fin_body(num_ref, w_ref, b_ref, out_ref):
    s = jnp.sum(num_ref[...], axis=0)
    h = s[:, :H] / (s[:, H:H + 1] + EPS)
    z = jnp.dot(h, w_ref[...], preferred_element_type=jnp.float32) + b_ref[...]
    lane = lax.broadcasted_iota(jnp.int32, z.shape, 1)
    mask = lane < C
    zm = jnp.where(mask, z, -jnp.inf)
    m = jnp.max(zm, axis=1, keepdims=True)
    e = jnp.where(mask, jnp.exp(z - m), 0.0)
    se = jnp.sum(e, axis=1, keepdims=True)
    out_ref[...] = z - m - jnp.log(se)


_fin = pl.pallas_call(
    _fin_body,
    grid=(NP // ROWS_BLK,),
    in_specs=[
        pl.BlockSpec((NPART, ROWS_BLK, HW), lambda i: (0, i, 0)),
        pl.BlockSpec((H, 128), lambda i: (0, 0)),
        pl.BlockSpec((1, 128), lambda i: (0, 0)),
    ],
    out_specs=pl.BlockSpec((ROWS_BLK, 128), lambda i: (i, 0)),
    out_shape=jax.ShapeDtypeStruct((NP, 128), jnp.float32),
)


@functools.cache
def _make_gather_sc(ngroups):
  gpt = ngroups // (NSC * NTILE)
  ne = ngroups * GROUP
  mesh = plsc.VectorSubcoreMesh(
      core_axis_name="c", subcore_axis_name="s",
      num_cores=NSC, num_subcores=NTILE)

  @functools.partial(
    pl.kernel,
    out_type=(
        jax.ShapeDtypeStruct((ne, H), jnp.float32),
        jax.ShapeDtypeStruct((ne, H), jnp.float32),
    ),
    mesh=mesh,
    compiler_params=pltpu.CompilerParams(use_tc_tiling_on_sc=False),
    scratch_types=[
        pltpu.VMEM((gpt, 2 * GROUP), jnp.int32),
        pltpu.VMEM((NBUF, GROUP, H), jnp.float32),
        pltpu.VMEM((NBUF, GROUP, H), jnp.float32),
        pltpu.SemaphoreType.DMA,
        pltpu.SemaphoreType.DMA,
    ],
  )
  def _gather_sc(t2, idxg, hs_o, hd_o, idxv, hsb, hdb, sg, sc_):
    c = lax.axis_index("c")
    s = lax.axis_index("s")
    g0 = (c * NTILE + s) * gpt
    pltpu.sync_copy(idxg.at[pl.ds(g0, gpt)], idxv)

    gh = [None] * gpt
    co = [None] * gpt

    def drain(g):
        b = g % NBUF
        e0 = (g0 + g) * GROUP
        gh[g][0].wait()
        gh[g][1].wait()
        co[g] = (
            pltpu.async_copy(hsb.at[b], hs_o.at[pl.ds(e0, GROUP)], sc_),
            pltpu.async_copy(hdb.at[b], hd_o.at[pl.ds(e0, GROUP)], sc_),
        )

    for g in range(gpt):
        b = g % NBUF
        if g >= NBUF:
            co[g - NBUF][0].wait()
            co[g - NBUF][1].wait()
        gh[g] = (
            pltpu.async_copy(t2.at[idxv.at[g, pl.ds(0, GROUP)]],
                             hsb.at[b], sg),
            pltpu.async_copy(t2.at[idxv.at[g, pl.ds(GROUP, GROUP)]],
                             hdb.at[b], sg),
        )
        if g >= 1:
            drain(g - 1)
    drain(gpt - 1)
    for g in range(max(0, gpt - NBUF), gpt):
        co[g][0].wait()
        co[g][1].wait()

  return _gather_sc


@functools.cache
def _make_scatter_sc(ngroups):
  gpt = ngroups // (NSC * NTILE)
  mesh = plsc.VectorSubcoreMesh(
      core_axis_name="c", subcore_axis_name="s",
      num_cores=NSC, num_subcores=NTILE)

  @functools.partial(
    pl.kernel,
    out_type=jax.ShapeDtypeStruct((NSC, NP, HW), jnp.float32),
    mesh=mesh,
    compiler_params=pltpu.CompilerParams(use_tc_tiling_on_sc=False),
    scratch_types=[
        pltpu.VMEM((gpt, GROUP), jnp.int32),
        pltpu.VMEM((NBUF, GROUP, HW), jnp.float32),
        pltpu.VMEM_SHARED((NP, HW), jnp.float32),
        pltpu.SemaphoreType.DMA,
    ],
  )
  def _scatter_sc(vals, dstg, z2, num_out, dstv, valv, acc, sl):
    c = lax.axis_index("c")
    s = lax.axis_index("s")
    r0 = s * ROWS_TILE
    pltpu.sync_copy(z2.at[pl.ds(r0, ROWS_TILE)], acc.at[pl.ds(r0, ROWS_TILE)])
    g0 = (c * NTILE + s) * gpt
    pltpu.sync_copy(dstg.at[pl.ds(g0, gpt)], dstv)
    plsc.subcore_barrier()

    ld = [None] * gpt

    def load(g):
        e0 = (g0 + g) * GROUP
        ld[g] = pltpu.async_copy(
            vals.at[pl.ds(e0, GROUP)], valv.at[g % NBUF], sl)

    for g in range(min(NBUF, gpt)):
        load(g)
    for g in range(gpt):
        ld[g].wait()
        pltpu.sync_copy(valv.at[g % NBUF], acc.at[dstv.at[g]], add=True)
        if g + NBUF < gpt:
            load(g + NBUF)
    plsc.subcore_barrier()
    pltpu.sync_copy(acc.at[pl.ds(r0, ROWS_TILE)],
                    num_out.at[c, pl.ds(r0, ROWS_TILE)])

  return _scatter_sc


def _prop(t2, idxp, dstp, betab, z2):
    gch = NGROUPS // NCHUNK
    ech = gch * GROUP
    gather = _make_gather_sc(gch)
    edge = _make_edge(gch)
    scatter = _make_scatter_sc(gch)
    parts = []
    for k in range(NCHUNK):
        ip = lax.slice_in_dim(idxp, k * gch, (k + 1) * gch)
        dp = lax.slice_in_dim(dstp, k * gch, (k + 1) * gch)
        hs, hd = gather(t2, ip)
        vals = edge(hs.reshape(ech // 4, 4 * H), hd.reshape(ech // 4, 4 * H),
                    betab)
        parts.append(scatter(vals.reshape(ech, HW), dp, z2))
    return jnp.concatenate(parts, axis=0)


def kernel(data_x, data_edge_index, W1, b1, W2, b2, beta2):
    src = data_edge_index[0]
    dst = data_edge_index[1]
    srcp = jnp.concatenate(
        [src, jnp.zeros((EP - E,), jnp.int32)]).reshape(NGROUPS, GROUP)
    dstp = jnp.concatenate(
        [dst, jnp.full((EP - E,), PAD_DST, jnp.int32)]).reshape(NGROUPS, GROUP)
    xp = jnp.pad(data_x, ((0, NP - N), (0, 0)))
    w1t = W1.T
    b1r = b1.reshape(1, H)
    w2t = jnp.pad(W2.T, ((0, 0), (0, 128 - C)))
    b2r = jnp.pad(b2, (0, 128 - C)).reshape(1, 128)
    z2 = jnp.zeros((NP, HW), jnp.float32)
    beta1b = jnp.ones((1, 128), jnp.float32)
    beta2b = jnp.broadcast_to(beta2.astype(jnp.float32), (1, 128))
    idxp = jnp.concatenate([srcp, dstp + NP], axis=1)

    t1 = _pre(xp, w1t, b1r)
    num = _prop(t1.reshape(2 * NP, H), idxp, dstp, beta1b, z2)
    t2 = _mid(num)
    num2 = _prop(t2.reshape(2 * NP, H), idxp, dstp, beta2b, z2)
    outp = _fin(num2, w2t, b2r)
    return outp[:N, :C]


# speedup vs baseline: 1.0008x; 1.0008x over previous
import functools

import jax
import jax.numpy as jnp
from jax import lax
from jax.experimental import pallas as pl
from jax.experimental.pallas import tpu as pltpu
from jax.experimental.pallas import tpu_sc as plsc

N = 10000
E = 320000
D = 128
H = 16
C = 7

NP = 10240
GROUP = 128
NSC = 2
NTILE = 16
NGROUPS = 2560
EP = NGROUPS * GROUP
GP_TILE = NGROUPS // (NSC * NTILE)
ROWS_TILE = NP // NTILE
PAD_DST = NP - 8
ROWS_BLK = 1024
EPS = 1e-16
HW = 32
GB = 16
NBUF = 8
NCHUNK = 2
NPART = NCHUNK * NSC
EB = GB * GROUP // 4


def _pre_body(x_ref, w_ref, b_ref, t_ref):
    h = jnp.dot(x_ref[...], w_ref[...], preferred_element_type=jnp.float32)
    h = jnp.maximum(h + b_ref[...], 0.0)
    nrm = jnp.maximum(jnp.sqrt(jnp.sum(h * h, axis=1)), 1e-12)
    t_ref[0] = h
    t_ref[1] = h / nrm[:, None]


_pre = pl.pallas_call(
    _pre_body,
    grid=(NP // ROWS_BLK,),
    in_specs=[
        pl.BlockSpec((ROWS_BLK, D), lambda i: (i, 0)),
        pl.BlockSpec((D, H), lambda i: (0, 0)),
        pl.BlockSpec((1, H), lambda i: (0, 0)),
    ],
    out_specs=pl.BlockSpec((2, ROWS_BLK, H), lambda i: (0, i, 0)),
    out_shape=jax.ShapeDtypeStruct((2, NP, H), jnp.float32),
)


def _edge_body(hs_ref, hd_ref, beta_ref, out_ref):
    hs = hs_ref[...]
    hd = hd_ref[...]
    bi = lax.broadcasted_iota(jnp.int32, (4 * H, 8 * H), 0)
    bl = lax.broadcasted_iota(jnp.int32, (4 * H, 8 * H), 1)
    seg = (bi // H) == (bl // HW)
    bd = jnp.where(seg, 1.0, 0.0)
    mm = jnp.where(seg & ((bi % H) == (bl % HW)), 1.0, 0.0)
    dot = jnp.dot(hs * hd, bd, preferred_element_type=jnp.float32)
    nsq = jnp.dot(hs * hs, bd, preferred_element_type=jnp.float32)
    nsrc = jnp.maximum(jnp.sqrt(nsq), 1e-12)
    ex = jnp.exp(beta_ref[0, 0] * dot / nsrc)
    hs128 = jnp.dot(hs, mm, preferred_element_type=jnp.float32)
    lane = lax.broadcasted_iota(jnp.int32, ex.shape, 1) % HW
    out_ref[...] = jnp.where(lane < H, hs128 * ex,
                             jnp.where(lane == H, ex, 0.0))


@functools.cache
def _make_edge(ngroups):
    ne = ngroups * GROUP
    return pl.pallas_call(
        _edge_body,
        grid=(ngroups // GB,),
        in_specs=[
            pl.BlockSpec((EB, 4 * H), lambda i: (i, 0)),
            pl.BlockSpec((EB, 4 * H), lambda i: (i, 0)),
            pl.BlockSpec((1, 128), lambda i: (0, 0)),
        ],
        out_specs=pl.BlockSpec((EB, 4 * HW), lambda i: (i, 0)),
        out_shape=jax.ShapeDtypeStruct((ne // 4, 4 * HW), jnp.float32),
    )


def _mid_body(num_ref, t_ref):
    s = jnp.sum(num_ref[...], axis=0)
    h = s[:, :H] / (s[:, H:H + 1] + EPS)
    nrm = jnp.maximum(jnp.sqrt(jnp.sum(h * h, axis=1)), 1e-12)
    t_ref[0] = h
    t_ref[1] = h / nrm[:, None]


_mid = pl.pallas_call(
    _mid_body,
    grid=(NP // ROWS_BLK,),
    in_specs=[
        pl.BlockSpec((NPART, ROWS_BLK, HW), lambda i: (0, i, 0)),
    ],
    out_specs=pl.BlockSpec((2, ROWS_BLK, H), lambda i: (0, i, 0)),
    out_shape=jax.ShapeDtypeStruct((2, NP, H), jnp.float32),
)


def _fin_body(num_ref, w_ref, b_ref, out_ref):
    s = jnp.sum(num_ref[...], axis=0)
    h = s[:, :H] / (s[:, H:H + 1] + EPS)
    z = jnp.dot(h, w_ref[...], preferred_element_type=jnp.float32) + b_ref[...]
    lane = lax.broadcasted_iota(jnp.int32, z.shape, 1)
    mask = lane < C
    zm = jnp.where(mask, z, -jnp.inf)
    m = jnp.max(zm, axis=1, keepdims=True)
    e = jnp.where(mask, jnp.exp(z - m), 0.0)
    se = jnp.sum(e, axis=1, keepdims=True)
    out_ref[...] = z - m - jnp.log(se)


_fin = pl.pallas_call(
    _fin_body,
    grid=(NP // ROWS_BLK,),
    in_specs=[
        pl.BlockSpec((NPART, ROWS_BLK, HW), lambda i: (0, i, 0)),
        pl.BlockSpec((H, 128), lambda i: (0, 0)),
        pl.BlockSpec((1, 128), lambda i: (0, 0)),
    ],
    out_specs=pl.BlockSpec((ROWS_BLK, 128), lambda i: (i, 0)),
    out_shape=jax.ShapeDtypeStruct((NP, 128), jnp.float32),
)


@functools.cache
def _make_gather_sc(ngroups):
  gpt = ngroups // (NSC * NTILE)
  ne = ngroups * GROUP
  mesh = plsc.VectorSubcoreMesh(
      core_axis_name="c", subcore_axis_name="s",
      num_cores=NSC, num_subcores=NTILE)

  @functools.partial(
    pl.kernel,
    out_type=(
        jax.ShapeDtypeStruct((ne, H), jnp.float32),
        jax.ShapeDtypeStruct((ne, H), jnp.float32),
    ),
    mesh=mesh,
    compiler_params=pltpu.CompilerParams(use_tc_tiling_on_sc=False),
    scratch_types=[
        pltpu.VMEM((gpt, 2 * GROUP), jnp.int32),
        pltpu.VMEM((NBUF, GROUP, H), jnp.float32),
        pltpu.VMEM((NBUF, GROUP, H), jnp.float32),
        pltpu.SemaphoreType.DMA,
        pltpu.SemaphoreType.DMA,
    ],
  )
  def _gather_sc(t2, idxg, hs_o, hd_o, idxv, hsb, hdb, sg, sc_):
    c = lax.axis_index("c")
    s = lax.axis_index("s")
    g0 = (c * NTILE + s) * gpt
    pltpu.sync_copy(idxg.at[pl.ds(g0, gpt)], idxv)

    gh = [None] * gpt
    co = [None] * gpt

    def drain(g):
        b = g % NBUF
        e0 = (g0 + g) * GROUP
        gh[g][0].wait()
        gh[g][1].wait()
        co[g] = (
            pltpu.async_copy(hsb.at[b], hs_o.at[pl.ds(e0, GROUP)], sc_),
            pltpu.async_copy(hdb.at[b], hd_o.at[pl.ds(e0, GROUP)], sc_),
        )

    for g in range(gpt):
        b = g % NBUF
        if g >= NBUF:
            co[g - NBUF][0].wait()
            co[g - NBUF][1].wait()
        gh[g] = (
            pltpu.async_copy(t2.at[idxv.at[g, pl.ds(0, GROUP)]],
                             hsb.at[b], sg),
            pltpu.async_copy(t2.at[idxv.at[g, pl.ds(GROUP, GROUP)]],
                             hdb.at[b], sg),
        )
        if g >= 1:
            drain(g - 1)
    drain(gpt - 1)
    for g in range(max(0, gpt - NBUF), gpt):
        co[g][0].wait()
        co[g][1].wait()

  return _gather_sc


@functools.cache
def _make_scatter_sc(ngroups):
  gpt = ngroups // (NSC * NTILE)
  mesh = plsc.VectorSubcoreMesh(
      core_axis_name="c", subcore_axis_name="s",
      num_cores=NSC, num_subcores=NTILE)

  @functools.partial(
    pl.kernel,
    out_type=jax.ShapeDtypeStruct((NSC, NP, HW), jnp.float32),
    mesh=mesh,
    compiler_params=pltpu.CompilerParams(use_tc_tiling_on_sc=False),
    scratch_types=[
        pltpu.VMEM((gpt, GROUP), jnp.int32),
        pltpu.VMEM((NBUF, GROUP, HW), jnp.float32),
        pltpu.VMEM_SHARED((NP, HW), jnp.float32),
        pltpu.SemaphoreType.DMA,
    ],
  )
  def _scatter_sc(vals, dstg, z2, num_out, dstv, valv, acc, sl):
    c = lax.axis_index("c")
    s = lax.axis_index("s")
    r0 = s * ROWS_TILE
    pltpu.sync_copy(z2.at[pl.ds(r0, ROWS_TILE)], acc.at[pl.ds(r0, ROWS_TILE)])
    g0 = (c * NTILE + s) * gpt
    pltpu.sync_copy(dstg.at[pl.ds(g0, gpt)], dstv)
    plsc.subcore_barrier()

    ld = [None] * gpt

    def load(g):
        e0 = (g0 + g) * GROUP
        ld[g] = pltpu.async_copy(
            vals.at[pl.ds(e0, GROUP)], valv.at[g % NBUF], sl)

    for g in range(min(NBUF, gpt)):
        load(g)
    for g in range(gpt):
        ld[g].wait()
        pltpu.sync_copy(valv.at[g % NBUF], acc.at[dstv.at[g]], add=True)
        if g + NBUF < gpt:
            load(g + NBUF)
    plsc.subcore_barrier()
    pltpu.sync_copy(acc.at[pl.ds(r0, ROWS_TILE)],
                    num_out.at[c, pl.ds(r0, ROWS_TILE)])

  return _scatter_sc


def _prop(t2, idxp, dstp, betab, z2):
    gch = NGROUPS // NCHUNK
    ech = gch * GROUP
    gather = _make_gather_sc(gch)
    edge = _make_edge(gch)
    scatter = _make_scatter_sc(gch)
    parts = []
    for k in range(NCHUNK):
        ip = lax.slice_in_dim(idxp, k * gch, (k + 1) * gch)
        dp = lax.slice_in_dim(dstp, k * gch, (k + 1) * gch)
        hs, hd = gather(t2, ip)
        vals = edge(hs.reshape(ech // 4, 4 * H), hd.reshape(ech // 4, 4 * H),
                    betab)
        parts.append(scatter(vals.reshape(ech, HW), dp, z2))
    return jnp.concatenate(parts, axis=0)


def kernel(data_x, data_edge_index, W1, b1, W2, b2, beta2):
    src = data_edge_index[0]
    dst = data_edge_index[1]
    srcp = jnp.concatenate(
        [src, jnp.zeros((EP - E,), jnp.int32)]).reshape(NGROUPS, GROUP)
    dstp = jnp.concatenate(
        [dst, jnp.full((EP - E,), PAD_DST, jnp.int32)]).reshape(NGROUPS, GROUP)
    xp = jnp.pad(data_x, ((0, NP - N), (0, 0)))
    w1t = W1.T
    b1r = b1.reshape(1, H)
    w2t = jnp.pad(W2.T, ((0, 0), (0, 128 - C)))
    b2r = jnp.pad(b2, (0, 128 - C)).reshape(1, 128)
    z2 = jnp.zeros((NP, HW), jnp.float32)
    beta1b = jnp.ones((1, 128), jnp.float32)
    beta2b = jnp.broadcast_to(beta2.astype(jnp.float32), (1, 128))
    idxp = jnp.concatenate([srcp, dstp + NP], axis=1)

    t1 = _pre(xp, w1t, b1r)
    num = _prop(t1.reshape(2 * NP, H), idxp, dstp, beta1b, z2)
    t2 = _mid(num)
    num2 = _prop(t2.reshape(2 * NP, H), idxp, dstp, beta2b, z2)
    outp = _fin(num2, w2t, b2r)
    return outp[:N, :C]


# speedup vs baseline: 1.0052x; 1.0044x over previous
import functools

import jax
import jax.numpy as jnp
from jax import lax
from jax.experimental import pallas as pl
from jax.experimental.pallas import tpu as pltpu
from jax.experimental.pallas import tpu_sc as plsc

N = 10000
E = 320000
D = 128
H = 16
C = 7

NP = 10240
GROUP = 128
NSC = 2
NTILE = 16
NGROUPS = 2560
EP = NGROUPS * GROUP
GP_TILE = NGROUPS // (NSC * NTILE)
ROWS_TILE = NP // NTILE
PAD_DST = NP - 8
ROWS_BLK = 1024
EPS = 1e-16
HW = 32
GB = 16
NBUF = 4
NCHUNK = 2
NPART = NCHUNK * NSC
EB = GB * GROUP // 4


def _pre_body(x_ref, w_ref, b_ref, t_ref):
    h = jnp.dot(x_ref[...], w_ref[...], preferred_element_type=jnp.float32)
    h = jnp.maximum(h + b_ref[...], 0.0)
    nrm = jnp.maximum(jnp.sqrt(jnp.sum(h * h, axis=1)), 1e-12)
    t_ref[0] = h
    t_ref[1] = h / nrm[:, None]


_pre = pl.pallas_call(
    _pre_body,
    grid=(NP // ROWS_BLK,),
    in_specs=[
        pl.BlockSpec((ROWS_BLK, D), lambda i: (i, 0)),
        pl.BlockSpec((D, H), lambda i: (0, 0)),
        pl.BlockSpec((1, H), lambda i: (0, 0)),
    ],
    out_specs=pl.BlockSpec((2, ROWS_BLK, H), lambda i: (0, i, 0)),
    out_shape=jax.ShapeDtypeStruct((2, NP, H), jnp.float32),
)


def _edge_body(hs_ref, hd_ref, beta_ref, out_ref):
    hs = hs_ref[...]
    hd = hd_ref[...]
    bi = lax.broadcasted_iota(jnp.int32, (4 * H, 8 * H), 0)
    bl = lax.broadcasted_iota(jnp.int32, (4 * H, 8 * H), 1)
    seg = (bi // H) == (bl // HW)
    bd = jnp.where(seg, 1.0, 0.0)
    mm = jnp.where(seg & ((bi % H) == (bl % HW)), 1.0, 0.0)
    dot = jnp.dot(hs * hd, bd, preferred_element_type=jnp.float32)
    nsq = jnp.dot(hs * hs, bd, preferred_element_type=jnp.float32)
    nsrc = jnp.maximum(jnp.sqrt(nsq), 1e-12)
    ex = jnp.exp(beta_ref[0, 0] * dot / nsrc)
    hs128 = jnp.dot(hs, mm, preferred_element_type=jnp.float32)
    lane = lax.broadcasted_iota(jnp.int32, ex.shape, 1) % HW
    out_ref[...] = jnp.where(lane < H, hs128 * ex,
                             jnp.where(lane == H, ex, 0.0))


@functools.cache
def _make_edge(ngroups):
    ne = ngroups * GROUP
    return pl.pallas_call(
        _edge_body,
        grid=(ngroups // GB,),
        in_specs=[
            pl.BlockSpec((EB, 4 * H), lambda i: (i, 0)),
            pl.BlockSpec((EB, 4 * H), lambda i: (i, 0)),
            pl.BlockSpec((1, 128), lambda i: (0, 0)),
        ],
        out_specs=pl.BlockSpec((EB, 4 * HW), lambda i: (i, 0)),
        out_shape=jax.ShapeDtypeStruct((ne // 4, 4 * HW), jnp.float32),
    )


def _mid_body(num_ref, t_ref):
    s = jnp.sum(num_ref[...], axis=0)
    h = s[:, :H] / (s[:, H:H + 1] + EPS)
    nrm = jnp.maximum(jnp.sqrt(jnp.sum(h * h, axis=1)), 1e-12)
    t_ref[0] = h
    t_ref[1] = h / nrm[:, None]


_mid = pl.pallas_call(
    _mid_body,
    grid=(NP // ROWS_BLK,),
    in_specs=[
        pl.BlockSpec((NPART, ROWS_BLK, HW), lambda i: (0, i, 0)),
    ],
    out_specs=pl.BlockSpec((2, ROWS_BLK, H), lambda i: (0, i, 0)),
    out_shape=jax.ShapeDtypeStruct((2, NP, H), jnp.float32),
)


def _fin_body(num_ref, w_ref, b_ref, out_ref):
    s = jnp.sum(num_ref[...], axis=0)
    h = s[:, :H] / (s[:, H:H + 1] + EPS)
    z = jnp.dot(h, w_ref[...], preferred_element_type=jnp.float32) + b_ref[...]
    lane = lax.broadcasted_iota(jnp.int32, z.shape, 1)
    mask = lane < C
    zm = jnp.where(mask, z, -jnp.inf)
    m = jnp.max(zm, axis=1, keepdims=True)
    e = jnp.where(mask, jnp.exp(z - m), 0.0)
    se = jnp.sum(e, axis=1, keepdims=True)
    out_ref[...] = z - m - jnp.log(se)


_fin = pl.pallas_call(
    _fin_body,
    grid=(NP // ROWS_BLK,),
    in_specs=[
        pl.BlockSpec((NPART, ROWS_BLK, HW), lambda i: (0, i, 0)),
        pl.BlockSpec((H, 128), lambda i: (0, 0)),
        pl.BlockSpec((1, 128), lambda i: (0, 0)),
    ],
    out_specs=pl.BlockSpec((ROWS_BLK, 128), lambda i: (i, 0)),
    out_shape=jax.ShapeDtypeStruct((NP, 128), jnp.float32),
)


@functools.cache
def _make_gather_sc(ngroups):
  gpt = ngroups // (NSC * NTILE)
  ne = ngroups * GROUP
  mesh = plsc.VectorSubcoreMesh(
      core_axis_name="c", subcore_axis_name="s",
      num_cores=NSC, num_subcores=NTILE)

  @functools.partial(
    pl.kernel,
    out_type=(
        jax.ShapeDtypeStruct((ne, H), jnp.float32),
        jax.ShapeDtypeStruct((ne, H), jnp.float32),
    ),
    mesh=mesh,
    compiler_params=pltpu.CompilerParams(use_tc_tiling_on_sc=False),
    scratch_types=[
        pltpu.VMEM((gpt, 2 * GROUP), jnp.int32),
        pltpu.VMEM((NBUF, GROUP, H), jnp.float32),
        pltpu.VMEM((NBUF, GROUP, H), jnp.float32),
        pltpu.SemaphoreType.DMA,
        pltpu.SemaphoreType.DMA,
    ],
  )
  def _gather_sc(t2, idxg, hs_o, hd_o, idxv, hsb, hdb, sg, sc_):
    c = lax.axis_index("c")
    s = lax.axis_index("s")
    g0 = (c * NTILE + s) * gpt
    pltpu.sync_copy(idxg.at[pl.ds(g0, gpt)], idxv)

    gh = [None] * gpt
    co = [None] * gpt

    def drain(g):
        b = g % NBUF
        e0 = (g0 + g) * GROUP
        gh[g][0].wait()
        gh[g][1].wait()
        co[g] = (
            pltpu.async_copy(hsb.at[b], hs_o.at[pl.ds(e0, GROUP)], sc_),
            pltpu.async_copy(hdb.at[b], hd_o.at[pl.ds(e0, GROUP)], sc_),
        )

    for g in range(gpt):
        b = g % NBUF
        if g >= NBUF:
            co[g - NBUF][0].wait()
            co[g - NBUF][1].wait()
        gh[g] = (
            pltpu.async_copy(t2.at[idxv.at[g, pl.ds(0, GROUP)]],
                             hsb.at[b], sg),
            pltpu.async_copy(t2.at[idxv.at[g, pl.ds(GROUP, GROUP)]],
                             hdb.at[b], sg),
        )
        if g >= 1:
            drain(g - 1)
    drain(gpt - 1)
    for g in range(max(0, gpt - NBUF), gpt):
        co[g][0].wait()
        co[g][1].wait()

  return _gather_sc


@functools.cache
def _make_scatter_sc(ngroups):
  gpt = ngroups // (NSC * NTILE)
  mesh = plsc.VectorSubcoreMesh(
      core_axis_name="c", subcore_axis_name="s",
      num_cores=NSC, num_subcores=NTILE)

  @functools.partial(
    pl.kernel,
    out_type=jax.ShapeDtypeStruct((NSC, NP, HW), jnp.float32),
    mesh=mesh,
    compiler_params=pltpu.CompilerParams(use_tc_tiling_on_sc=False),
    scratch_types=[
        pltpu.VMEM((gpt, GROUP), jnp.int32),
        pltpu.VMEM((NBUF, GROUP, HW), jnp.float32),
        pltpu.VMEM_SHARED((NP, HW), jnp.float32),
        pltpu.SemaphoreType.DMA,
    ],
  )
  def _scatter_sc(vals, dstg, z2, num_out, dstv, valv, acc, sl):
    c = lax.axis_index("c")
    s = lax.axis_index("s")
    r0 = s * ROWS_TILE
    pltpu.sync_copy(z2.at[pl.ds(r0, ROWS_TILE)], acc.at[pl.ds(r0, ROWS_TILE)])
    g0 = (c * NTILE + s) * gpt
    pltpu.sync_copy(dstg.at[pl.ds(g0, gpt)], dstv)
    plsc.subcore_barrier()

    ld = [None] * gpt

    def load(g):
        e0 = (g0 + g) * GROUP
        ld[g] = pltpu.async_copy(
            vals.at[pl.ds(e0, GROUP)], valv.at[g % NBUF], sl)

    for g in range(min(NBUF, gpt)):
        load(g)
    for g in range(gpt):
        ld[g].wait()
        pltpu.sync_copy(valv.at[g % NBUF], acc.at[dstv.at[g]], add=True)
        if g + NBUF < gpt:
            load(g + NBUF)
    plsc.subcore_barrier()
    pltpu.sync_copy(acc.at[pl.ds(r0, ROWS_TILE)],
                    num_out.at[c, pl.ds(r0, ROWS_TILE)])

  return _scatter_sc


def _prop(t2, idxp, dstp, betab, z2):
    gch = NGROUPS // NCHUNK
    ech = gch * GROUP
    gather = _make_gather_sc(gch)
    edge = _make_edge(gch)
    scatter = _make_scatter_sc(gch)
    parts = []
    for k in range(NCHUNK):
        ip = lax.slice_in_dim(idxp, k * gch, (k + 1) * gch)
        dp = lax.slice_in_dim(dstp, k * gch, (k + 1) * gch)
        hs, hd = gather(t2, ip)
        vals = edge(hs.reshape(ech // 4, 4 * H), hd.reshape(ech // 4, 4 * H),
                    betab)
        parts.append(scatter(vals.reshape(ech, HW), dp, z2))
    return jnp.concatenate(parts, axis=0)


def kernel(data_x, data_edge_index, W1, b1, W2, b2, beta2):
    src = data_edge_index[0]
    dst = data_edge_index[1]
    srcp = jnp.concatenate(
        [src, jnp.zeros((EP - E,), jnp.int32)]).reshape(NGROUPS, GROUP)
    dstp = jnp.concatenate(
        [dst, jnp.full((EP - E,), PAD_DST, jnp.int32)]).reshape(NGROUPS, GROUP)
    xp = jnp.pad(data_x, ((0, NP - N), (0, 0)))
    w1t = W1.T
    b1r = b1.reshape(1, H)
    w2t = jnp.pad(W2.T, ((0, 0), (0, 128 - C)))
    b2r = jnp.pad(b2, (0, 128 - C)).reshape(1, 128)
    z2 = jnp.zeros((NP, HW), jnp.float32)
    beta1b = jnp.ones((1, 128), jnp.float32)
    beta2b = jnp.broadcast_to(beta2.astype(jnp.float32), (1, 128))
    idxp = jnp.concatenate([srcp, dstp + NP], axis=1)

    t1 = _pre(xp, w1t, b1r)
    num = _prop(t1.reshape(2 * NP, H), idxp, dstp, beta1b, z2)
    t2 = _mid(num)
    num2 = _prop(t2.reshape(2 * NP, H), idxp, dstp, beta2b, z2)
    outp = _fin(num2, w2t, b2r)
    return outp[:N, :C]
